# baseline (speedup 1.0000x reference)
import jax
import jax.numpy as jnp
from jax.experimental import pallas as pl
from jax.experimental.pallas import tpu as pltpu

_N = 4096
_EPS2 = 1.9 ** 2
_MIN_SAMPLES = 3
_N_ITERS = 20
_BLK = 256
_NBLK = _N // _BLK
_NF = float(_N)
_HI = jax.lax.Precision.HIGHEST


def _dbscan_body(inp_ref, inpT_ref, seg_ref, segT_ref,
                 labels_out, gath_out, feat_out,
                 adj_ref, core_col_ref, core_row_ref, lcol_ref, lrow_ref):
    f32 = jnp.float32

    st = segT_ref[...]
    stmax = jnp.max(st, axis=0, keepdims=True)
    krow = jax.lax.broadcasted_iota(jnp.int32, st.shape, 0).astype(f32)
    sem_row = jnp.min(jnp.where(st == stmax, krow, 5.0), axis=0, keepdims=True)

    b_row = inpT_ref[0:1, :]
    x_row = inpT_ref[1:2, :]
    y_row = inpT_ref[2:3, :]
    z_row = inpT_ref[3:4, :]

    def build(i, cnt_row):
        ds = pl.ds(i * _BLK, _BLK)
        dx = inp_ref[ds, 1:2] - x_row
        dy = inp_ref[ds, 2:3] - y_row
        dz = inp_ref[ds, 3:4] - z_row
        d2 = dx * dx + dy * dy + dz * dz
        sb = seg_ref[ds, :]
        sbm = jnp.max(sb, axis=1, keepdims=True)
        kb = jax.lax.broadcasted_iota(jnp.int32, sb.shape, 1).astype(f32)
        sem_b = jnp.min(jnp.where(sb == sbm, kb, 5.0), axis=1, keepdims=True)
        adj = (d2 <= _EPS2) & (inp_ref[ds, 0:1] == b_row) & (sem_b == sem_row)
        adj_ref[ds, :] = adj.astype(jnp.int8)
        adj_f = adj.astype(f32)
        cnt_col = jnp.sum(adj_f, axis=1, keepdims=True)
        core_col_ref[ds, :] = (cnt_col >= _MIN_SAMPLES).astype(f32)
        return cnt_row + jnp.sum(adj_f, axis=0, keepdims=True)

    cnt_row = jax.lax.fori_loop(0, _NBLK, build, jnp.zeros((1, _N), f32))
    core_row_ref[...] = (cnt_row >= _MIN_SAMPLES).astype(f32)

    lcol_ref[...] = jax.lax.broadcasted_iota(jnp.int32, (_N, 1), 0).astype(f32)
    lrow_ref[...] = jax.lax.broadcasted_iota(jnp.int32, (1, _N), 1).astype(f32)

    def pair(state):
        t, _ = state

        tr = jnp.where(core_row_ref[...] > 0.5, lrow_ref[...], _NF) - _NF

        def blk_a(i, ch):
            ds = pl.ds(i * _BLK, _BLK)
            af = adj_ref[ds, :].astype(f32)
            colmin = jnp.min(af * tr + _NF, axis=1, keepdims=True)
            old = lcol_ref[ds, :]
            new = jnp.minimum(old, colmin)
            lcol_ref[ds, :] = new
            return jnp.maximum(ch, jnp.max(old - new))

        ch_a = jax.lax.fori_loop(0, _NBLK, blk_a, f32(0.0))

        def blk_b(i, racc):
            ds = pl.ds(i * _BLK, _BLK)
            af = adj_ref[ds, :].astype(f32)
            tc = jnp.where(core_col_ref[ds, :] > 0.5, lcol_ref[ds, :],
                           _NF) - _NF
            return jnp.minimum(racc, jnp.min(af * tc + _NF, axis=0,
                                             keepdims=True))

        racc = jax.lax.fori_loop(0, _NBLK, blk_b,
                                 jnp.full((1, _N), _NF, f32))
        old_row = lrow_ref[...]
        new_row = jnp.minimum(old_row, racc)
        lrow_ref[...] = new_row
        ch_b = jnp.max(old_row - new_row)
        return t + 1, jnp.maximum(ch_a, ch_b) > 0.0

    jax.lax.while_loop(lambda s: (s[0] < _N_ITERS // 2) & s[1], pair,
                       (0, True))

    st2 = seg_ref[...]
    s2max = jnp.max(st2, axis=1, keepdims=True)
    kcol = jax.lax.broadcasted_iota(jnp.int32, st2.shape, 1).astype(f32)
    sem_col = jnp.min(jnp.where(st2 == s2max, kcol, 5.0), axis=1, keepdims=True)
    gath_out[...] = sem_col
    jcol = jax.lax.broadcasted_iota(jnp.int32, (_N, 1), 0).astype(f32)

    def final(i, facc):
        ds = pl.ds(i * _BLK, _BLK)
        ohT = (jcol == lrow_ref[:, ds]).astype(f32)
        return facc + jnp.dot(ohT, inp_ref[ds, 1:5], precision=_HI)

    facc = jax.lax.fori_loop(0, _NBLK, final, jnp.zeros((_N, 4), f32))
    feat_out[...] = facc
    labels_out[...] = lrow_ref[...].astype(jnp.int32)


@jax.jit
def kernel(input, segmentation):
    inp = input.astype(jnp.float32)
    seg = segmentation.astype(jnp.float32)
    labels, gath, feats = pl.pallas_call(
        _dbscan_body,
        out_shape=[
            jax.ShapeDtypeStruct((1, _N), jnp.int32),
            jax.ShapeDtypeStruct((_N, 1), jnp.float32),
            jax.ShapeDtypeStruct((_N, 4), jnp.float32),
        ],
        scratch_shapes=[
            pltpu.VMEM((_N, _N), jnp.int8),
            pltpu.VMEM((_N, 1), jnp.float32),
            pltpu.VMEM((1, _N), jnp.float32),
            pltpu.VMEM((_N, 1), jnp.float32),
            pltpu.VMEM((1, _N), jnp.float32),
        ],
    )(inp, inp.T, seg, seg.T)
    labels = labels[0, :]
    frag_batch = inp[:, 0].astype(jnp.int32)
    frag_seg = gath[:, 0].astype(jnp.int32)
    return labels, frag_batch, frag_seg, feats


# speedup vs baseline: 1.1055x; 1.1055x over previous
import jax
import jax.numpy as jnp
from jax.experimental import pallas as pl
from jax.experimental.pallas import tpu as pltpu

_N = 4096
_EPS2 = 1.9 ** 2
_MIN_SAMPLES = 3
_N_ITERS = 20
_BLK = 256
_NBLK = _N // _BLK
_NF = float(_N)
_HI = jax.lax.Precision.HIGHEST


def _dbscan_body(inp_ref, inpT_ref, seg_ref, segT_ref,
                 labels_out, gath_out, feat_out,
                 adj_ref, core_col_ref, core_row_ref, lcol_ref, lrow_ref):
    f32 = jnp.float32

    st = segT_ref[...]
    stmax = jnp.max(st, axis=0, keepdims=True)
    krow = jax.lax.broadcasted_iota(jnp.int32, st.shape, 0).astype(f32)
    sem_row = jnp.min(jnp.where(st == stmax, krow, 5.0), axis=0, keepdims=True)

    b_row = inpT_ref[0:1, :]
    x_row = inpT_ref[1:2, :]
    y_row = inpT_ref[2:3, :]
    z_row = inpT_ref[3:4, :]

    def build(i, cnt_row):
        ds = pl.ds(i * _BLK, _BLK)
        dx = inp_ref[ds, 1:2] - x_row
        dy = inp_ref[ds, 2:3] - y_row
        dz = inp_ref[ds, 3:4] - z_row
        d2 = dx * dx + dy * dy + dz * dz
        sb = seg_ref[ds, :]
        sbm = jnp.max(sb, axis=1, keepdims=True)
        kb = jax.lax.broadcasted_iota(jnp.int32, sb.shape, 1).astype(f32)
        sem_b = jnp.min(jnp.where(sb == sbm, kb, 5.0), axis=1, keepdims=True)
        adj = (d2 <= _EPS2) & (inp_ref[ds, 0:1] == b_row) & (sem_b == sem_row)
        adj_ref[ds, :] = adj.astype(jnp.int8)
        adj_f = adj.astype(f32)
        cnt_col = jnp.sum(adj_f, axis=1, keepdims=True)
        core_col_ref[ds, :] = (cnt_col >= _MIN_SAMPLES).astype(f32)
        return cnt_row + jnp.sum(adj_f, axis=0, keepdims=True)

    cnt_row = jax.lax.fori_loop(0, _NBLK, build, jnp.zeros((1, _N), f32))
    core_row_ref[...] = (cnt_row >= _MIN_SAMPLES).astype(f32)

    lcol_ref[...] = jax.lax.broadcasted_iota(jnp.int32, (_N, 1), 0).astype(f32)
    lrow_ref[...] = jax.lax.broadcasted_iota(jnp.int32, (1, _N), 1).astype(f32)

    def one_iter(state):
        t, _ = state
        tr = jnp.where(core_row_ref[...] > 0.5, lrow_ref[...], _NF) - _NF

        def blk(i, racc):
            ds = pl.ds(i * _BLK, _BLK)
            af = adj_ref[ds, :].astype(f32)
            lcol_blk = lcol_ref[ds, :]
            tc = jnp.where(core_col_ref[ds, :] > 0.5, lcol_blk, _NF) - _NF
            colmin = jnp.min(af * tr + _NF, axis=1, keepdims=True)
            lcol_ref[ds, :] = jnp.minimum(lcol_blk, colmin)
            rowmin = jnp.min(af * tc + _NF, axis=0, keepdims=True)
            return jnp.minimum(racc, rowmin)

        racc = jax.lax.fori_loop(0, _NBLK, blk, jnp.full((1, _N), _NF, f32))
        old_row = lrow_ref[...]
        new_row = jnp.minimum(old_row, racc)
        lrow_ref[...] = new_row
        return t + 1, jnp.max(old_row - new_row) > 0.0

    jax.lax.while_loop(lambda s: (s[0] < _N_ITERS) & s[1], one_iter,
                       (0, True))

    st2 = seg_ref[...]
    s2max = jnp.max(st2, axis=1, keepdims=True)
    kcol = jax.lax.broadcasted_iota(jnp.int32, st2.shape, 1).astype(f32)
    sem_col = jnp.min(jnp.where(st2 == s2max, kcol, 5.0), axis=1, keepdims=True)
    gath_out[...] = sem_col
    jcol = jax.lax.broadcasted_iota(jnp.int32, (_N, 1), 0).astype(f32)

    def final(i, facc):
        ds = pl.ds(i * _BLK, _BLK)
        ohT = (jcol == lrow_ref[:, ds]).astype(f32)
        return facc + jnp.dot(ohT, inp_ref[ds, 1:5], precision=_HI)

    facc = jax.lax.fori_loop(0, _NBLK, final, jnp.zeros((_N, 4), f32))
    feat_out[...] = facc
    labels_out[...] = lrow_ref[...].astype(jnp.int32)


@jax.jit
def kernel(input, segmentation):
    inp = input.astype(jnp.float32)
    seg = segmentation.astype(jnp.float32)
    labels, gath, feats = pl.pallas_call(
        _dbscan_body,
        out_shape=[
            jax.ShapeDtypeStruct((1, _N), jnp.int32),
            jax.ShapeDtypeStruct((_N, 1), jnp.float32),
            jax.ShapeDtypeStruct((_N, 4), jnp.float32),
        ],
        scratch_shapes=[
            pltpu.VMEM((_N, _N), jnp.int8),
            pltpu.VMEM((_N, 1), jnp.float32),
            pltpu.VMEM((1, _N), jnp.float32),
            pltpu.VMEM((_N, 1), jnp.float32),
            pltpu.VMEM((1, _N), jnp.float32),
        ],
    )(inp, inp.T, seg, seg.T)
    labels = labels[0, :]
    frag_batch = inp[:, 0].astype(jnp.int32)
    frag_seg = gath[:, 0].astype(jnp.int32)
    return labels, frag_batch, frag_seg, feats


# speedup vs baseline: 1.1440x; 1.0348x over previous
import jax
import jax.numpy as jnp
from jax.experimental import pallas as pl
from jax.experimental.pallas import tpu as pltpu

_N = 4096
_EPS2 = 1.9 ** 2
_MIN_SAMPLES = 3
_N_ITERS = 20
_BLK = 256
_NBLK = _N // _BLK
_NF = float(_N)
_HI = jax.lax.Precision.HIGHEST


def _dbscan_body(inp_ref, inpT_ref, seg_ref, segT_ref,
                 labels_out, gath_out, feat_out,
                 adj_ref, core_col_ref, core_row_ref, lcol_ref, lrow_ref):
    f32 = jnp.float32

    st = segT_ref[...]
    stmax = jnp.max(st, axis=0, keepdims=True)
    krow = jax.lax.broadcasted_iota(jnp.int32, st.shape, 0).astype(f32)
    sem_row = jnp.min(jnp.where(st == stmax, krow, 5.0), axis=0, keepdims=True)

    key_row = inpT_ref[0:1, :] * 8.0 + sem_row
    x_row = inpT_ref[1:2, :]
    y_row = inpT_ref[2:3, :]
    z_row = inpT_ref[3:4, :]

    def build(i, cnt_row):
        ds = pl.ds(i * _BLK, _BLK)
        dx = inp_ref[ds, 1:2] - x_row
        dy = inp_ref[ds, 2:3] - y_row
        dz = inp_ref[ds, 3:4] - z_row
        d2 = dx * dx + dy * dy + dz * dz
        sb = seg_ref[ds, :]
        sbm = jnp.max(sb, axis=1, keepdims=True)
        kb = jax.lax.broadcasted_iota(jnp.int32, sb.shape, 1).astype(f32)
        sem_b = jnp.min(jnp.where(sb == sbm, kb, 5.0), axis=1, keepdims=True)
        key_b = inp_ref[ds, 0:1] * 8.0 + sem_b
        adj = (d2 <= _EPS2) & (key_b == key_row)
        adj_ref[ds, :] = adj.astype(jnp.int8)
        adj_f = adj.astype(f32)
        cnt_col = jnp.sum(adj_f, axis=1, keepdims=True)
        core_col_ref[ds, :] = (cnt_col >= _MIN_SAMPLES).astype(f32)
        return cnt_row + jnp.sum(adj_f, axis=0, keepdims=True)

    cnt_row = jax.lax.fori_loop(0, _NBLK, build, jnp.zeros((1, _N), f32))
    core_row_ref[...] = (cnt_row >= _MIN_SAMPLES).astype(f32)

    lcol_ref[...] = jax.lax.broadcasted_iota(jnp.int32, (_N, 1), 0).astype(f32)
    lrow_ref[...] = jax.lax.broadcasted_iota(jnp.int32, (1, _N), 1).astype(f32)

    def one_iter(state):
        t, _ = state
        tr = jnp.where(core_row_ref[...] > 0.5, lrow_ref[...], _NF) - _NF

        def blk(i, racc):
            ds = pl.ds(i * _BLK, _BLK)
            af = adj_ref[ds, :].astype(f32)
            lcol_blk = lcol_ref[ds, :]
            tc = jnp.where(core_col_ref[ds, :] > 0.5, lcol_blk, _NF) - _NF
            colmin = jnp.min(af * tr + _NF, axis=1, keepdims=True)
            lcol_ref[ds, :] = jnp.minimum(lcol_blk, colmin)
            rowmin = jnp.min(af * tc + _NF, axis=0, keepdims=True)
            return jnp.minimum(racc, rowmin)

        racc = jax.lax.fori_loop(0, _NBLK, blk, jnp.full((1, _N), _NF, f32))
        old_row = lrow_ref[...]
        new_row = jnp.minimum(old_row, racc)
        lrow_ref[...] = new_row
        return t + 1, jnp.max(old_row - new_row) > 0.0

    jax.lax.while_loop(lambda s: (s[0] < _N_ITERS) & s[1], one_iter,
                       (0, True))

    st2 = seg_ref[...]
    s2max = jnp.max(st2, axis=1, keepdims=True)
    kcol = jax.lax.broadcasted_iota(jnp.int32, st2.shape, 1).astype(f32)
    sem_col = jnp.min(jnp.where(st2 == s2max, kcol, 5.0), axis=1, keepdims=True)
    gath_out[...] = sem_col
    jcol = jax.lax.broadcasted_iota(jnp.int32, (_N, 1), 0).astype(f32)

    def final(i, facc):
        ds = pl.ds(i * _BLK, _BLK)
        ohT = (jcol == lrow_ref[:, ds]).astype(f32)
        return facc + jnp.dot(ohT, inp_ref[ds, 1:5], precision=_HI)

    facc = jax.lax.fori_loop(0, _NBLK, final, jnp.zeros((_N, 4), f32))
    feat_out[...] = facc
    labels_out[...] = lrow_ref[...].astype(jnp.int32)


@jax.jit
def kernel(input, segmentation):
    inp = input.astype(jnp.float32)
    seg = segmentation.astype(jnp.float32)
    labels, gath, feats = pl.pallas_call(
        _dbscan_body,
        out_shape=[
            jax.ShapeDtypeStruct((1, _N), jnp.int32),
            jax.ShapeDtypeStruct((_N, 1), jnp.float32),
            jax.ShapeDtypeStruct((_N, 4), jnp.float32),
        ],
        scratch_shapes=[
            pltpu.VMEM((_N, _N), jnp.int8),
            pltpu.VMEM((_N, 1), jnp.float32),
            pltpu.VMEM((1, _N), jnp.float32),
            pltpu.VMEM((_N, 1), jnp.float32),
            pltpu.VMEM((1, _N), jnp.float32),
        ],
    )(inp, inp.T, seg, seg.T)
    labels = labels[0, :]
    frag_batch = inp[:, 0].astype(jnp.int32)
    frag_seg = gath[:, 0].astype(jnp.int32)
    return labels, frag_batch, frag_seg, feats
